# baseline (speedup 1.0000x reference)
import functools

import jax
import jax.numpy as jnp
from jax import lax
from jax.experimental import pallas as pl
from jax.experimental.pallas import tpu as pltpu
from jax.experimental.pallas import tpu_sc as plsc

N_NODES = 10000
NPAD = 10240
N_EDGES = 320000
D_IN = 128
D_HID = 32
D_OUT = 10
DW = 128

NC = 2
NS = 16
NW = NC * NS
CHUNK = 64
K = 160
NB = NW * K
EP = NB * CHUNK
NBUF = 4
PIECE = 64
CORE_PIECES = (1, 4)


def _sc_scatter_rows(idx2d, g):
    mesh = plsc.VectorSubcoreMesh(
        core_axis_name="c", subcore_axis_name="s", num_cores=NC, num_subcores=NS
    )
    zeros = jnp.zeros((NPAD, DW), jnp.float32)

    @functools.partial(
        pl.kernel,
        out_type=jax.ShapeDtypeStruct((NC, NPAD, DW), jnp.float32),
        mesh=mesh,
        scratch_types=[
            pltpu.VMEM((PIECE, 2 * CHUNK), jnp.int32),
            pltpu.VMEM((NBUF, CHUNK, DW), jnp.float32),
            pltpu.VMEM_SHARED((NPAD, DW), jnp.float32),
            [pltpu.SemaphoreType.DMA] * NBUF,
        ],
    )
    def body(idx_hbm, g_hbm, z_hbm, out_hbm, idx_v, rows_v, acc, sems):
        cid = lax.axis_index("c")
        sid = lax.axis_index("s")
        zrows = NPAD // NS
        rsl = pl.ds(sid * zrows, zrows)
        pltpu.sync_copy(z_hbm.at[rsl], acc.at[rsl])
        plsc.subcore_barrier()

        def gather_start(c, b):
            pltpu.async_copy(
                g_hbm.at[idx_v.at[c, pl.ds(0, CHUNK)]], rows_v.at[b], sems[b]
            )

        def gather_wait(b):
            pltpu.make_async_copy(
                g_hbm.at[pl.ds(0, CHUNK)], rows_v.at[b], sems[b]
            ).wait()

        def scatter(c, p):
            gather_wait(p)
            pltpu.sync_copy(
                rows_v.at[p], acc.at[idx_v.at[c, pl.ds(CHUNK, CHUNK)]], add=True
            )

        body_iters = (PIECE - NBUF - 1) // NBUF

        def run_piece(base):
            pltpu.sync_copy(idx_hbm.at[pl.ds(base, PIECE)], idx_v)

            for b in range(NBUF - 1):
                gather_start(b, b)

            def step(m, carry):
                for p in range(NBUF):
                    c = m * NBUF + p
                    gather_start(c + NBUF - 1, (p + NBUF - 1) % NBUF)
                    scatter(c, p)
                return carry

            lax.fori_loop(0, body_iters, step, 0)

            for c in range(body_iters * NBUF, PIECE):
                if c + NBUF - 1 < PIECE:
                    gather_start(c + NBUF - 1, (c + NBUF - 1) % NBUF)
                scatter(c, c % NBUF)

        n0 = NS * CORE_PIECES[0] * PIECE

        @pl.when(cid == 0)
        def _():
            for q in range(CORE_PIECES[0]):
                run_piece(sid * (CORE_PIECES[0] * PIECE) + q * PIECE)

        @pl.when(cid == 1)
        def _():
            for q in range(CORE_PIECES[1]):
                run_piece(n0 + sid * (CORE_PIECES[1] * PIECE) + q * PIECE)

        plsc.subcore_barrier()
        pltpu.sync_copy(acc.at[rsl], out_hbm.at[cid].at[rsl])

    return body(idx2d, g, zeros)


def _sc_degree(dst2d):
    mesh = plsc.VectorSubcoreMesh(
        core_axis_name="c", subcore_axis_name="s", num_cores=NC, num_subcores=NS
    )
    zeros = jnp.zeros((NPAD,), jnp.float32)

    @functools.partial(
        pl.kernel,
        out_type=jax.ShapeDtypeStruct((NC, NPAD), jnp.float32),
        mesh=mesh,
        scratch_types=[
            pltpu.VMEM((K, CHUNK), jnp.int32),
            pltpu.VMEM((CHUNK,), jnp.float32),
            pltpu.VMEM_SHARED((NPAD,), jnp.float32),
        ],
    )
    def body(dst_hbm, z_hbm, out_hbm, dst_v, ones_v, acc):
        cid = lax.axis_index("c")
        sid = lax.axis_index("s")
        wid = sid * NC + cid
        zrows = NPAD // NS
        rsl = pl.ds(sid * zrows, zrows)
        pltpu.sync_copy(z_hbm.at[rsl], acc.at[rsl])
        for i in range(CHUNK // 16):
            ones_v[pl.ds(i * 16, 16)] = jnp.ones((16,), jnp.float32)
        pltpu.sync_copy(dst_hbm.at[pl.ds(wid * K, K)], dst_v)
        plsc.subcore_barrier()

        def step(j, carry):
            pltpu.sync_copy(ones_v, acc.at[dst_v.at[j]], add=True)
            return carry

        lax.fori_loop(0, K, step, 0)

        plsc.subcore_barrier()
        pltpu.sync_copy(acc.at[rsl], out_hbm.at[cid].at[rsl])

    return body(dst2d, zeros)


def _tc_layer1(x_p, W1p, degp):
    def body(x_ref, w_ref, degp_ref, dis_ref, g_ref):
        deg = 1.0 + degp_ref[0] + degp_ref[1]
        dis = lax.rsqrt(deg)
        h = jnp.dot(x_ref[...], w_ref[...], preferred_element_type=jnp.float32)
        dis_ref[...] = dis
        g_ref[...] = dis * h

    return pl.pallas_call(
        body,
        out_shape=(
            jax.ShapeDtypeStruct((NPAD, 1), jnp.float32),
            jax.ShapeDtypeStruct((NPAD, DW), jnp.float32),
        ),
    )(x_p, W1p, degp)


def _tc_layer2(accp, g1, dis, b1p, W2p):
    def body(a_ref, g_ref, dis_ref, b_ref, w_ref, g2_ref):
        z = dis_ref[...] * (a_ref[0] + a_ref[1] + g_ref[...]) + b_ref[...]
        z = jnp.maximum(z, 0.0)
        h2 = jnp.dot(z, w_ref[...], preferred_element_type=jnp.float32)
        g2_ref[...] = dis_ref[...] * h2

    return pl.pallas_call(
        body,
        out_shape=jax.ShapeDtypeStruct((NPAD, DW), jnp.float32),
    )(accp, g1, dis, b1p, W2p)


def _tc_combine(accp, g2, dis, b2p):
    def body(a_ref, g_ref, dis_ref, b_ref, out_ref):
        out_ref[...] = dis_ref[...] * (a_ref[0] + a_ref[1] + g_ref[...]) + b_ref[...]

    return pl.pallas_call(
        body,
        out_shape=jax.ShapeDtypeStruct((NPAD, DW), jnp.float32),
    )(accp, g2, dis, b2p)


def kernel(x, edge_index, W1, b1, W2, b2):
    src = edge_index[0].astype(jnp.int32)
    dst = edge_index[1].astype(jnp.int32)
    pad = EP - N_EDGES
    src_p = jnp.concatenate([src, jnp.zeros((pad,), jnp.int32)]).reshape(NB, CHUNK)
    dst_p = jnp.concatenate([dst, jnp.full((pad,), N_NODES, jnp.int32)]).reshape(NB, CHUNK)
    idx_p = jnp.concatenate([src_p, dst_p], axis=1)

    degp = _sc_degree(dst_p)[..., None]

    x_p = jnp.pad(x, ((0, NPAD - N_NODES), (0, 0)))
    W1p = jnp.pad(W1, ((0, 0), (0, DW - D_HID)))
    W2p = jnp.pad(W2, ((0, DW - D_HID), (0, DW - D_OUT)))
    b1p = jnp.pad(b1, (0, DW - D_HID))
    b2p = jnp.pad(b2, (0, DW - D_OUT))

    dis, g1 = _tc_layer1(x_p, W1p, degp)
    acc1 = _sc_scatter_rows(idx_p, g1)

    g2 = _tc_layer2(acc1, g1, dis, b1p, W2p)
    acc2 = _sc_scatter_rows(idx_p, g2)

    out = _tc_combine(acc2, g2, dis, b2p)
    return out[:N_NODES, :D_OUT]


# speedup vs baseline: 1.0826x; 1.0826x over previous
import functools

import jax
import jax.numpy as jnp
from jax import lax
from jax.experimental import pallas as pl
from jax.experimental.pallas import tpu as pltpu
from jax.experimental.pallas import tpu_sc as plsc

N_NODES = 10000
NPAD = 10240
N_EDGES = 320000
D_IN = 128
D_HID = 32
D_OUT = 10
DW = 128

NC = 2
NS = 16
NW = NC * NS
CHUNK = 64
K = 160
NB = NW * K
EP = NB * CHUNK
NBUF = 4
PIECE = 64
CORE_PIECES = (4, 1)


def _sc_scatter_rows(idx2d, g):
    mesh = plsc.VectorSubcoreMesh(
        core_axis_name="c", subcore_axis_name="s", num_cores=NC, num_subcores=NS
    )
    zeros = jnp.zeros((NPAD, DW), jnp.float32)

    @functools.partial(
        pl.kernel,
        out_type=jax.ShapeDtypeStruct((NC, NPAD, DW), jnp.float32),
        mesh=mesh,
        scratch_types=[
            pltpu.VMEM((PIECE, 2 * CHUNK), jnp.int32),
            pltpu.VMEM((NBUF, CHUNK, DW), jnp.float32),
            pltpu.VMEM_SHARED((NPAD, DW), jnp.float32),
            [pltpu.SemaphoreType.DMA] * NBUF,
        ],
    )
    def body(idx_hbm, g_hbm, z_hbm, out_hbm, idx_v, rows_v, acc, sems):
        cid = lax.axis_index("c")
        sid = lax.axis_index("s")
        zrows = NPAD // NS
        rsl = pl.ds(sid * zrows, zrows)
        pltpu.sync_copy(z_hbm.at[rsl], acc.at[rsl])
        plsc.subcore_barrier()

        def gather_start(c, b):
            pltpu.async_copy(
                g_hbm.at[idx_v.at[c, pl.ds(0, CHUNK)]], rows_v.at[b], sems[b]
            )

        def gather_wait(b):
            pltpu.make_async_copy(
                g_hbm.at[pl.ds(0, CHUNK)], rows_v.at[b], sems[b]
            ).wait()

        def scatter(c, p):
            gather_wait(p)
            pltpu.sync_copy(
                rows_v.at[p], acc.at[idx_v.at[c, pl.ds(CHUNK, CHUNK)]], add=True
            )

        body_iters = (PIECE - NBUF - 1) // NBUF

        def run_piece(base):
            pltpu.sync_copy(idx_hbm.at[pl.ds(base, PIECE)], idx_v)

            for b in range(NBUF - 1):
                gather_start(b, b)

            def step(m, carry):
                for p in range(NBUF):
                    c = m * NBUF + p
                    gather_start(c + NBUF - 1, (p + NBUF - 1) % NBUF)
                    scatter(c, p)
                return carry

            lax.fori_loop(0, body_iters, step, 0)

            for c in range(body_iters * NBUF, PIECE):
                if c + NBUF - 1 < PIECE:
                    gather_start(c + NBUF - 1, (c + NBUF - 1) % NBUF)
                scatter(c, c % NBUF)

        n0 = NS * CORE_PIECES[0] * PIECE

        @pl.when(cid == 0)
        def _():
            for q in range(CORE_PIECES[0]):
                run_piece(sid * (CORE_PIECES[0] * PIECE) + q * PIECE)

        @pl.when(cid == 1)
        def _():
            for q in range(CORE_PIECES[1]):
                run_piece(n0 + sid * (CORE_PIECES[1] * PIECE) + q * PIECE)

        plsc.subcore_barrier()
        pltpu.sync_copy(acc.at[rsl], out_hbm.at[cid].at[rsl])

    return body(idx2d, g, zeros)


def _sc_degree(dst2d):
    mesh = plsc.VectorSubcoreMesh(
        core_axis_name="c", subcore_axis_name="s", num_cores=NC, num_subcores=NS
    )
    zeros = jnp.zeros((NPAD,), jnp.float32)

    @functools.partial(
        pl.kernel,
        out_type=jax.ShapeDtypeStruct((NC, NPAD), jnp.float32),
        mesh=mesh,
        scratch_types=[
            pltpu.VMEM((K, CHUNK), jnp.int32),
            pltpu.VMEM((CHUNK,), jnp.float32),
            pltpu.VMEM_SHARED((NPAD,), jnp.float32),
        ],
    )
    def body(dst_hbm, z_hbm, out_hbm, dst_v, ones_v, acc):
        cid = lax.axis_index("c")
        sid = lax.axis_index("s")
        wid = sid * NC + cid
        zrows = NPAD // NS
        rsl = pl.ds(sid * zrows, zrows)
        pltpu.sync_copy(z_hbm.at[rsl], acc.at[rsl])
        for i in range(CHUNK // 16):
            ones_v[pl.ds(i * 16, 16)] = jnp.ones((16,), jnp.float32)
        pltpu.sync_copy(dst_hbm.at[pl.ds(wid * K, K)], dst_v)
        plsc.subcore_barrier()

        def step(j, carry):
            pltpu.sync_copy(ones_v, acc.at[dst_v.at[j]], add=True)
            return carry

        lax.fori_loop(0, K, step, 0)

        plsc.subcore_barrier()
        pltpu.sync_copy(acc.at[rsl], out_hbm.at[cid].at[rsl])

    return body(dst2d, zeros)


def _tc_layer1(x_p, W1p, degp):
    def body(x_ref, w_ref, degp_ref, dis_ref, g_ref):
        deg = 1.0 + degp_ref[0] + degp_ref[1]
        dis = lax.rsqrt(deg)
        h = jnp.dot(x_ref[...], w_ref[...], preferred_element_type=jnp.float32)
        dis_ref[...] = dis
        g_ref[...] = dis * h

    return pl.pallas_call(
        body,
        out_shape=(
            jax.ShapeDtypeStruct((NPAD, 1), jnp.float32),
            jax.ShapeDtypeStruct((NPAD, DW), jnp.float32),
        ),
    )(x_p, W1p, degp)


def _tc_layer2(accp, g1, dis, b1p, W2p):
    def body(a_ref, g_ref, dis_ref, b_ref, w_ref, g2_ref):
        z = dis_ref[...] * (a_ref[0] + a_ref[1] + g_ref[...]) + b_ref[...]
        z = jnp.maximum(z, 0.0)
        h2 = jnp.dot(z, w_ref[...], preferred_element_type=jnp.float32)
        g2_ref[...] = dis_ref[...] * h2

    return pl.pallas_call(
        body,
        out_shape=jax.ShapeDtypeStruct((NPAD, DW), jnp.float32),
    )(accp, g1, dis, b1p, W2p)


def _tc_combine(accp, g2, dis, b2p):
    def body(a_ref, g_ref, dis_ref, b_ref, out_ref):
        out_ref[...] = dis_ref[...] * (a_ref[0] + a_ref[1] + g_ref[...]) + b_ref[...]

    return pl.pallas_call(
        body,
        out_shape=jax.ShapeDtypeStruct((NPAD, DW), jnp.float32),
    )(accp, g2, dis, b2p)


def kernel(x, edge_index, W1, b1, W2, b2):
    src = edge_index[0].astype(jnp.int32)
    dst = edge_index[1].astype(jnp.int32)
    pad = EP - N_EDGES
    src_p = jnp.concatenate([src, jnp.zeros((pad,), jnp.int32)]).reshape(NB, CHUNK)
    dst_p = jnp.concatenate([dst, jnp.full((pad,), N_NODES, jnp.int32)]).reshape(NB, CHUNK)
    idx_p = jnp.concatenate([src_p, dst_p], axis=1)

    degp = _sc_degree(dst_p)[..., None]

    x_p = jnp.pad(x, ((0, NPAD - N_NODES), (0, 0)))
    W1p = jnp.pad(W1, ((0, 0), (0, DW - D_HID)))
    W2p = jnp.pad(W2, ((0, DW - D_HID), (0, DW - D_OUT)))
    b1p = jnp.pad(b1, (0, DW - D_HID))
    b2p = jnp.pad(b2, (0, DW - D_OUT))

    dis, g1 = _tc_layer1(x_p, W1p, degp)
    acc1 = _sc_scatter_rows(idx_p, g1)

    g2 = _tc_layer2(acc1, g1, dis, b1p, W2p)
    acc2 = _sc_scatter_rows(idx_p, g2)

    out = _tc_combine(acc2, g2, dis, b2p)
    return out[:N_NODES, :D_OUT]


# speedup vs baseline: 1.0975x; 1.0138x over previous
import functools

import jax
import jax.numpy as jnp
from jax import lax
from jax.experimental import pallas as pl
from jax.experimental.pallas import tpu as pltpu
from jax.experimental.pallas import tpu_sc as plsc

N_NODES = 10000
NPAD = 10240
N_EDGES = 320000
D_IN = 128
D_HID = 32
D_OUT = 10
DW = 128

NC = 2
NS = 16
NW = NC * NS
CHUNK = 64
K = 160
NB = NW * K
EP = NB * CHUNK
NBUF = 4
PIECE = 64
CORE_PIECES = (4, 1)


def _sc_scatter_rows(idx2d, g):
    mesh = plsc.VectorSubcoreMesh(
        core_axis_name="c", subcore_axis_name="s", num_cores=NC, num_subcores=NS
    )
    zeros = jnp.zeros((NPAD, DW), jnp.float32)

    @functools.partial(
        pl.kernel,
        out_type=jax.ShapeDtypeStruct((NC, NPAD, DW), jnp.float32),
        mesh=mesh,
        scratch_types=[
            pltpu.VMEM((PIECE, 2 * CHUNK), jnp.int32),
            pltpu.VMEM((NBUF, CHUNK, DW), jnp.float32),
            pltpu.VMEM_SHARED((NPAD, DW), jnp.float32),
            [pltpu.SemaphoreType.DMA] * NBUF,
        ],
    )
    def body(idx_hbm, g_hbm, z_hbm, out_hbm, idx_v, rows_v, acc, sems):
        cid = lax.axis_index("c")
        sid = lax.axis_index("s")
        zrows = NPAD // NS
        rsl = pl.ds(sid * zrows, zrows)
        pltpu.sync_copy(z_hbm.at[rsl], acc.at[rsl])
        plsc.subcore_barrier()

        def gather_start(c, b):
            pltpu.async_copy(
                g_hbm.at[idx_v.at[c, pl.ds(0, CHUNK)]], rows_v.at[b], sems[b]
            )

        def gather_wait(b):
            pltpu.make_async_copy(
                g_hbm.at[pl.ds(0, CHUNK)], rows_v.at[b], sems[b]
            ).wait()

        def scatter(c, p):
            gather_wait(p)
            pltpu.sync_copy(
                rows_v.at[p], acc.at[idx_v.at[c, pl.ds(CHUNK, CHUNK)]], add=True
            )

        body_iters = (PIECE - NBUF - 1) // NBUF

        def run_piece(base):
            pltpu.sync_copy(idx_hbm.at[pl.ds(base, PIECE)], idx_v)

            for b in range(NBUF - 1):
                gather_start(b, b)

            def step(m, carry):
                for p in range(NBUF):
                    c = m * NBUF + p
                    gather_start(c + NBUF - 1, (p + NBUF - 1) % NBUF)
                    scatter(c, p)
                return carry

            lax.fori_loop(0, body_iters, step, 0)

            for c in range(body_iters * NBUF, PIECE):
                if c + NBUF - 1 < PIECE:
                    gather_start(c + NBUF - 1, (c + NBUF - 1) % NBUF)
                scatter(c, c % NBUF)

        n0 = NS * CORE_PIECES[0] * PIECE

        @pl.when(cid == 0)
        def _():
            for q in range(CORE_PIECES[0]):
                run_piece(sid * (CORE_PIECES[0] * PIECE) + q * PIECE)

        @pl.when(cid == 1)
        def _():
            for q in range(CORE_PIECES[1]):
                run_piece(n0 + sid * (CORE_PIECES[1] * PIECE) + q * PIECE)

        plsc.subcore_barrier()
        pltpu.sync_copy(acc.at[rsl], out_hbm.at[cid].at[rsl])

    return body(idx2d, g, zeros)


def _sc_degree(dst2d):
    mesh = plsc.VectorSubcoreMesh(
        core_axis_name="c", subcore_axis_name="s", num_cores=NC, num_subcores=NS
    )
    zeros = jnp.zeros((NPAD,), jnp.float32)

    @functools.partial(
        pl.kernel,
        out_type=jax.ShapeDtypeStruct((NC, NPAD), jnp.float32),
        mesh=mesh,
        scratch_types=[
            pltpu.VMEM((K, CHUNK), jnp.int32),
            pltpu.VMEM((CHUNK,), jnp.float32),
            pltpu.VMEM_SHARED((NPAD,), jnp.float32),
        ],
    )
    def body(dst_hbm, z_hbm, out_hbm, dst_v, ones_v, acc):
        cid = lax.axis_index("c")
        sid = lax.axis_index("s")
        wid = sid * NC + cid
        zrows = NPAD // NS
        rsl = pl.ds(sid * zrows, zrows)
        pltpu.sync_copy(z_hbm.at[rsl], acc.at[rsl])
        for i in range(CHUNK // 16):
            ones_v[pl.ds(i * 16, 16)] = jnp.ones((16,), jnp.float32)
        pltpu.sync_copy(dst_hbm.at[pl.ds(wid * K, K)], dst_v)
        plsc.subcore_barrier()

        def step(j, carry):
            pltpu.sync_copy(ones_v, acc.at[dst_v.at[j]], add=True)
            return carry

        lax.fori_loop(0, K, step, 0)

        plsc.subcore_barrier()
        pltpu.sync_copy(acc.at[rsl], out_hbm.at[cid].at[rsl])

    return body(dst2d, zeros)


def _tc_matmul1(x_p, W1p):
    def body(x_ref, w_ref, h_ref):
        h_ref[...] = jnp.dot(x_ref[...], w_ref[...], preferred_element_type=jnp.float32)

    return pl.pallas_call(
        body,
        out_shape=jax.ShapeDtypeStruct((NPAD, DW), jnp.float32),
    )(x_p, W1p)


def _tc_layer1(h1, degp):
    def body(h_ref, degp_ref, dis_ref, g_ref):
        deg = 1.0 + degp_ref[0] + degp_ref[1]
        dis = lax.rsqrt(deg)
        dis_ref[...] = dis
        g_ref[...] = dis * h_ref[...]

    return pl.pallas_call(
        body,
        out_shape=(
            jax.ShapeDtypeStruct((NPAD, 1), jnp.float32),
            jax.ShapeDtypeStruct((NPAD, DW), jnp.float32),
        ),
    )(h1, degp)


def _tc_layer2(accp, g1, dis, b1p, W2p):
    def body(a_ref, g_ref, dis_ref, b_ref, w_ref, g2_ref):
        z = dis_ref[...] * (a_ref[0] + a_ref[1] + g_ref[...]) + b_ref[...]
        z = jnp.maximum(z, 0.0)
        h2 = jnp.dot(z, w_ref[...], preferred_element_type=jnp.float32)
        g2_ref[...] = dis_ref[...] * h2

    return pl.pallas_call(
        body,
        out_shape=jax.ShapeDtypeStruct((NPAD, DW), jnp.float32),
    )(accp, g1, dis, b1p, W2p)


def _tc_combine(accp, g2, dis, b2p):
    def body(a_ref, g_ref, dis_ref, b_ref, out_ref):
        out_ref[...] = dis_ref[...] * (a_ref[0] + a_ref[1] + g_ref[...]) + b_ref[...]

    return pl.pallas_call(
        body,
        out_shape=jax.ShapeDtypeStruct((NPAD, DW), jnp.float32),
    )(accp, g2, dis, b2p)


def kernel(x, edge_index, W1, b1, W2, b2):
    src = edge_index[0].astype(jnp.int32)
    dst = edge_index[1].astype(jnp.int32)
    pad = EP - N_EDGES
    src_p = jnp.concatenate([src, jnp.zeros((pad,), jnp.int32)]).reshape(NB, CHUNK)
    dst_p = jnp.concatenate([dst, jnp.full((pad,), N_NODES, jnp.int32)]).reshape(NB, CHUNK)
    idx_p = jnp.concatenate([src_p, dst_p], axis=1)

    degp = _sc_degree(dst_p)[..., None]

    x_p = jnp.pad(x, ((0, NPAD - N_NODES), (0, 0)))
    W1p = jnp.pad(W1, ((0, 0), (0, DW - D_HID)))
    h1 = _tc_matmul1(x_p, W1p)
    W2p = jnp.pad(W2, ((0, DW - D_HID), (0, DW - D_OUT)))
    b1p = jnp.pad(b1, (0, DW - D_HID))
    b2p = jnp.pad(b2, (0, DW - D_OUT))

    dis, g1 = _tc_layer1(h1, degp)
    acc1 = _sc_scatter_rows(idx_p, g1)

    g2 = _tc_layer2(acc1, g1, dis, b1p, W2p)
    acc2 = _sc_scatter_rows(idx_p, g2)

    out = _tc_combine(acc2, g2, dis, b2p)
    return out[:N_NODES, :D_OUT]
